# baseline (speedup 1.0000x reference)
import functools

import jax
import jax.numpy as jnp
from jax import lax
from jax.experimental import pallas as pl
from jax.experimental.pallas import tpu as pltpu
from jax.experimental.pallas import tpu_sc as plsc

CHUNK = 128


def _tc_table(tab_t):
    d, v = tab_t.shape
    grid = (v + CHUNK - 1) // CHUNK

    def body(in_ref, out_ref):
        t = in_ref[...].T.reshape(d, 2, d)
        out_ref[...] = jnp.concatenate([t[:, 0, :], t[:, 1, :]], axis=1)

    return pl.pallas_call(
        body,
        grid=(grid,),
        in_specs=[pl.BlockSpec((d, CHUNK), lambda i: (0, i))],
        out_specs=pl.BlockSpec((d, 2 * d), lambda i: (i, 0)),
        out_shape=jax.ShapeDtypeStruct((v // 2, 2 * d), jnp.float32),
    )(tab_t)


def _tc_out(rows2):
    n = rows2.shape[0] * 128
    n_bt = n // (50 * 64 * 128)

    def body(in_ref, out_ref):
        x = in_ref[...].reshape(128, 25, 128)
        x = x.transpose(1, 0, 2)
        y = x.transpose(0, 2, 1)
        y = y.reshape(25, 16, 8, 128)
        y = y.reshape(400, 8, 128).reshape(50, 8, 8, 128)
        out_ref[...] = y.reshape(50, 8, 1, 8, 128)

    return pl.pallas_call(
        body,
        grid=(n_bt,),
        in_specs=[pl.BlockSpec((3200, 128), lambda i: (i, 0))],
        out_specs=pl.BlockSpec((50, 8, 1, 8, 128), lambda i: (0, 0, i, 0, 0)),
        out_shape=jax.ShapeDtypeStruct((50, 8, n_bt, 8, 128), jnp.float32),
    )(rows2)


@functools.lru_cache(maxsize=None)
def _make_gather(n_rows, rows_per_worker, d):
    mesh = plsc.VectorSubcoreMesh(core_axis_name="c", subcore_axis_name="s")
    info = plsc.get_sparse_core_info()
    nc = info.num_cores

    M = 8
    K = 4
    assert rows_per_worker % M == 0 and rows_per_worker > M + K

    @functools.partial(
        pl.kernel,
        mesh=mesh,
        out_type=jax.ShapeDtypeStruct((n_rows, CHUNK, d), jnp.float32),
        scratch_types=[
            pltpu.VMEM((rows_per_worker, CHUNK), jnp.int32),
            pltpu.VMEM((M, CHUNK, d), jnp.float32),
            pltpu.SemaphoreType.DMA((M,)),
            pltpu.SemaphoreType.DMA((M,)),
        ],
        compiler_params=pltpu.CompilerParams(use_tc_tiling_on_sc=False),
    )
    def k(idx_hbm, table_hbm, out_hbm, idx_v, rows_v, gsem, wsem):
        wid = lax.axis_index("s") * nc + lax.axis_index("c")
        base = wid * rows_per_worker
        pltpu.sync_copy(idx_hbm.at[pl.ds(base, rows_per_worker)], idx_v)

        def gather(j, b):
            pltpu.async_copy(table_hbm.at[idx_v.at[j]], rows_v.at[b],
                             gsem.at[b])

        def write(j, b):
            pltpu.make_async_copy(rows_v.at[b], out_hbm.at[base + j],
                                  gsem.at[b]).wait()
            pltpu.async_copy(rows_v.at[b], out_hbm.at[base + j], wsem.at[b])

        for j in range(M):
            gather(j, j % M)
            if j >= K:
                write(j - K, (j - K) % M)

        def group(g, carry):
            for b in range(M):
                j = g * M + b
                pltpu.make_async_copy(rows_v.at[b], out_hbm.at[0],
                                      wsem.at[b]).wait()
                gather(j, b)
                b2 = (b - K) % M
                write(j - K, b2)
            return carry

        lax.fori_loop(1, rows_per_worker // M, group, 0)

        for jj in range(rows_per_worker - K, rows_per_worker):
            write(jj, jj % M)
        for b in range(M):
            pltpu.make_async_copy(rows_v.at[b], out_hbm.at[0],
                                  wsem.at[b]).wait()

    return k


def kernel(input_text, table):
    b, h = input_text.shape
    v, d = table.shape
    total = b * h
    n_rows = total // CHUNK
    rows_per_worker = n_rows // 32
    idx = input_text.reshape(n_rows, CHUNK).astype(jnp.int32)
    table_lin = _tc_table(table.T).reshape(v, d)
    rows = _make_gather(n_rows, rows_per_worker, d)(idx, table_lin)
    out5 = _tc_out(rows.reshape(total * d // 128, 128))
    return out5.transpose(2, 4, 0, 1, 3).reshape(b, h, d)


# speedup vs baseline: 4.6997x; 4.6997x over previous
import functools

import jax
import jax.numpy as jnp
from jax import lax
from jax.experimental import pallas as pl
from jax.experimental.pallas import tpu as pltpu
from jax.experimental.pallas import tpu_sc as plsc

CHUNK = 128


def _tc_table(tab_t):
    d, v = tab_t.shape
    grid = (v + CHUNK - 1) // CHUNK

    def body(in_ref, out_ref):
        t = in_ref[...].T.reshape(d, 2, d)
        out_ref[...] = jnp.concatenate([t[:, 0, :], t[:, 1, :]], axis=1)

    return pl.pallas_call(
        body,
        grid=(grid,),
        in_specs=[pl.BlockSpec((d, CHUNK), lambda i: (0, i))],
        out_specs=pl.BlockSpec((d, 2 * d), lambda i: (i, 0)),
        out_shape=jax.ShapeDtypeStruct((v // 2, 2 * d), jnp.float32),
    )(tab_t)


def _tc_out(rows2):
    n = rows2.shape[0] * 128
    n_bt = n // (50 * 64 * 128)

    def body(in_ref, out_ref):
        x = in_ref[...].reshape(128, 25, 128)
        x = x.transpose(1, 0, 2)
        y = x.transpose(0, 2, 1)
        y = y.reshape(25, 16, 8, 128)
        y = y.reshape(400, 8, 128).reshape(50, 8, 8, 128)
        out_ref[...] = y.reshape(50, 8, 1, 8, 128)

    return pl.pallas_call(
        body,
        grid=(n_bt,),
        in_specs=[pl.BlockSpec((3200, 128), lambda i: (i, 0))],
        out_specs=pl.BlockSpec((50, 8, 1, 8, 128), lambda i: (0, 0, i, 0, 0)),
        out_shape=jax.ShapeDtypeStruct((50, 8, n_bt, 8, 128), jnp.float32),
    )(rows2)


@functools.lru_cache(maxsize=None)
def _make_gather(n_rows, rows_per_worker, d):
    mesh = plsc.VectorSubcoreMesh(core_axis_name="c", subcore_axis_name="s")
    info = plsc.get_sparse_core_info()
    nc = info.num_cores

    M = 8
    K = 4
    assert rows_per_worker % M == 0 and rows_per_worker > M + K

    @functools.partial(
        pl.kernel,
        mesh=mesh,
        out_type=jax.ShapeDtypeStruct((n_rows, CHUNK, d), jnp.float32),
        scratch_types=[
            pltpu.VMEM((rows_per_worker, CHUNK), jnp.int32),
            pltpu.VMEM((M, CHUNK, d), jnp.float32),
            pltpu.SemaphoreType.DMA((M,)),
            pltpu.SemaphoreType.DMA((M,)),
        ],
        compiler_params=pltpu.CompilerParams(use_tc_tiling_on_sc=False),
    )
    def k(idx_hbm, table_hbm, out_hbm, idx_v, rows_v, gsem, wsem):
        wid = lax.axis_index("s") * nc + lax.axis_index("c")
        base = wid * rows_per_worker
        pltpu.sync_copy(idx_hbm.at[pl.ds(base, rows_per_worker)], idx_v)

        def gather(j, b):
            pltpu.async_copy(table_hbm.at[idx_v.at[j]], rows_v.at[b],
                             gsem.at[b])

        def write(j, b):
            pltpu.make_async_copy(rows_v.at[b], out_hbm.at[base + j],
                                  gsem.at[b]).wait()
            pltpu.async_copy(rows_v.at[b], out_hbm.at[base + j], wsem.at[b])

        for j in range(M):
            gather(j, j % M)
            if j >= K:
                write(j - K, (j - K) % M)

        def group(g, carry):
            for b in range(M):
                j = g * M + b
                pltpu.make_async_copy(rows_v.at[b], out_hbm.at[0],
                                      wsem.at[b]).wait()
                gather(j, b)
                b2 = (b - K) % M
                write(j - K, b2)
            return carry

        lax.fori_loop(1, rows_per_worker // M, group, 0)

        for jj in range(rows_per_worker - K, rows_per_worker):
            write(jj, jj % M)
        for b in range(M):
            pltpu.make_async_copy(rows_v.at[b], out_hbm.at[0],
                                  wsem.at[b]).wait()

    return k


def kernel(input_text, table):
    b, h = input_text.shape
    v, d = table.shape
    total = b * h
    n_rows = total // CHUNK
    rows_per_worker = n_rows // 32
    idx = input_text.reshape(n_rows, CHUNK).astype(jnp.int32)
    rows = _make_gather(n_rows, rows_per_worker, d)(idx, table)
    out5 = _tc_out(rows.reshape(total * d // 128, 128))
    return out5.transpose(2, 4, 0, 1, 3).reshape(b, h, d)
